# baseline (speedup 1.0000x reference)
import functools

import jax
import jax.numpy as jnp
from jax import lax
from jax.experimental import pallas as pl
from jax.experimental.pallas import tpu as pltpu
from jax.experimental.pallas import tpu_sc as plsc

_BATCH = 16384
_DIM = 64
_NC = 2
_NS = 16
_NW = _NC * _NS
_ROWS_PER_W = _BATCH // _NW
_CHUNK = 128
_NCHUNK = _ROWS_PER_W // _CHUNK
_L = 16
_GROUPS = _ROWS_PER_W // _L


def _sc_body(user_idx_hbm, item_idx_hbm, user_table_hbm, item_table_hbm,
             out_hbm, idx_u, idx_i, rows_u, rows_i, out_v, sem):
    wid = lax.axis_index("s") * _NC + lax.axis_index("c")
    base = wid * _ROWS_PER_W

    pltpu.sync_copy(user_idx_hbm.at[wid], idx_u)
    pltpu.sync_copy(item_idx_hbm.at[wid], idx_i)

    copies = []
    for j in range(_NCHUNK):
        dst = rows_u.at[pl.ds(j * _CHUNK, _CHUNK), :]
        copies.append(pltpu.async_copy(user_table_hbm.at[idx_u.at[j]], dst, sem))
    for j in range(_NCHUNK):
        dst = rows_i.at[pl.ds(j * _CHUNK, _CHUNK), :]
        copies.append(pltpu.async_copy(item_table_hbm.at[idx_i.at[j]], dst, sem))
    for c in copies:
        c.wait()

    lane = lax.iota(jnp.int32, _L)
    perms = [lane ^ k for k in (1, 2, 4, 8)]
    gd = lax.GatherDimensionNumbers(
        offset_dims=(), collapsed_slice_dims=(0,), start_index_map=(0,))

    def shuffle(x, p):
        return lax.gather(x, p[:, None], gd, slice_sizes=(1,),
                          mode=lax.GatherScatterMode.PROMISE_IN_BOUNDS)

    def group_body(g, carry):
        res = jnp.zeros((_L,), jnp.float32)
        for j in range(_L):
            r = g * _L + j
            acc = rows_u[r, 0:_L] * rows_i[r, 0:_L]
            for c in range(1, _DIM // _L):
                acc = acc + rows_u[r, c * _L:(c + 1) * _L] * rows_i[r, c * _L:(c + 1) * _L]
            for p in perms:
                acc = acc + shuffle(acc, p)
            res = jnp.where(lane == j, acc, res)
        out_v[pl.ds(g * _L, _L)] = res
        return carry

    lax.fori_loop(0, _GROUPS, group_body, 0)

    pltpu.sync_copy(out_v, out_hbm.at[pl.ds(base, _ROWS_PER_W)])


@jax.jit
def _run(user_idx, item_idx, user_table, item_table):
    mesh = plsc.VectorSubcoreMesh(core_axis_name="c", subcore_axis_name="s")
    f = pl.kernel(
        _sc_body,
        mesh=mesh,
        compiler_params=pltpu.CompilerParams(use_tc_tiling_on_sc=False),
        out_type=jax.ShapeDtypeStruct((_BATCH,), jnp.float32),
        scratch_types=[
            pltpu.VMEM((_NCHUNK, _CHUNK), jnp.int32),
            pltpu.VMEM((_NCHUNK, _CHUNK), jnp.int32),
            pltpu.VMEM((_ROWS_PER_W, _DIM), jnp.float32),
            pltpu.VMEM((_ROWS_PER_W, _DIM), jnp.float32),
            pltpu.VMEM((_ROWS_PER_W,), jnp.float32),
            pltpu.SemaphoreType.DMA,
        ],
    )
    return f(user_idx, item_idx, user_table, item_table)


def kernel(inputs, user_table, item_table):
    user_idx = inputs[:, 0].reshape(_NW, _NCHUNK, _CHUNK)
    item_idx = inputs[:, 1].reshape(_NW, _NCHUNK, _CHUNK)
    return _run(user_idx, item_idx, user_table, item_table)


# speedup vs baseline: 1.5709x; 1.5709x over previous
import jax
import jax.numpy as jnp
from jax import lax
from jax.experimental import pallas as pl
from jax.experimental.pallas import tpu as pltpu
from jax.experimental.pallas import tpu_sc as plsc

_BATCH = 16384
_DIM = 64
_NC = 2
_NS = 16
_NW = _NC * _NS
_ROWS_PER_W = _BATCH // _NW
_L = 16
_GROUPS = _ROWS_PER_W // _L


def _sc_body(idx_u_hbm, idx_i_hbm, table_u_hbm, table_i_hbm, out_hbm,
             idx_u, idx_i, rows, out_v, sem):
    wid = lax.axis_index("s") * _NC + lax.axis_index("c")
    base = wid * _ROWS_PER_W

    pltpu.sync_copy(idx_u_hbm.at[wid], idx_u)
    pltpu.sync_copy(idx_i_hbm.at[wid], idx_i)

    lane = lax.iota(jnp.int32, _L)
    zero = jnp.zeros((_L,), jnp.int32)

    def fire(g, carry):
        vu = idx_u[pl.ds(g * _L, _L)]
        vi = idx_i[pl.ds(g * _L, _L)]
        for j in range(_L):
            ru = lax.reduce_sum(jnp.where(lane == j, vu, zero), axes=(0,))
            ri = lax.reduce_sum(jnp.where(lane == j, vi, zero), axes=(0,))
            r = g * _L + j
            pltpu.async_copy(table_u_hbm.at[ru], rows.at[r, pl.ds(0, _DIM)], sem)
            pltpu.async_copy(table_i_hbm.at[ri], rows.at[r, pl.ds(_DIM, _DIM)], sem)
        return carry

    lax.fori_loop(0, _GROUPS, fire, 0)

    def drain(j, carry):
        pltpu.make_async_copy(
            table_u_hbm.at[0], rows.at[j, pl.ds(0, _DIM)], sem).wait()
        pltpu.make_async_copy(
            table_i_hbm.at[0], rows.at[j, pl.ds(_DIM, _DIM)], sem).wait()
        return carry

    lax.fori_loop(0, _ROWS_PER_W, drain, 0)

    perms = [lane ^ k for k in (1, 2, 4, 8)]
    gd = lax.GatherDimensionNumbers(
        offset_dims=(), collapsed_slice_dims=(0,), start_index_map=(0,))

    def shuffle(x, p):
        return lax.gather(x, p[:, None], gd, slice_sizes=(1,),
                          mode=lax.GatherScatterMode.PROMISE_IN_BOUNDS)

    def group_body(g, carry):
        res = jnp.zeros((_L,), jnp.float32)
        for j in range(_L):
            r = g * _L + j
            acc = rows[r, 0:_L] * rows[r, _DIM:_DIM + _L]
            for c in range(1, _DIM // _L):
                acc = acc + rows[r, c * _L:(c + 1) * _L] * rows[r, _DIM + c * _L:_DIM + (c + 1) * _L]
            for p in perms:
                acc = acc + shuffle(acc, p)
            res = jnp.where(lane == j, acc, res)
        out_v[pl.ds(g * _L, _L)] = res
        return carry

    lax.fori_loop(0, _GROUPS, group_body, 0)

    pltpu.sync_copy(out_v, out_hbm.at[pl.ds(base, _ROWS_PER_W)])


@jax.jit
def _run(idx_u, idx_i, table_u, table_i):
    mesh = plsc.VectorSubcoreMesh(core_axis_name="c", subcore_axis_name="s")
    f = pl.kernel(
        _sc_body,
        mesh=mesh,
        compiler_params=pltpu.CompilerParams(needs_layout_passes=False),
        out_type=jax.ShapeDtypeStruct((_BATCH,), jnp.float32),
        scratch_types=[
            pltpu.VMEM((_ROWS_PER_W,), jnp.int32),
            pltpu.VMEM((_ROWS_PER_W,), jnp.int32),
            pltpu.VMEM((_ROWS_PER_W, 2 * _DIM), jnp.float32),
            pltpu.VMEM((_ROWS_PER_W,), jnp.float32),
            pltpu.SemaphoreType.DMA,
        ],
    )
    return f(idx_u, idx_i, table_u, table_i)


def kernel(inputs, user_table, item_table):
    user_idx = inputs[:, 0].reshape(_NW, _ROWS_PER_W)
    item_idx = inputs[:, 1].reshape(_NW, _ROWS_PER_W)
    return _run(user_idx, item_idx, user_table, item_table)
